# Initial kernel scaffold; baseline (speedup 1.0000x reference)
import jax
import jax.numpy as jnp
from jax.experimental import pallas as pl


def kernel(input, mask, source):
    raise NotImplementedError("write your pallas kernel here")


# speedup vs baseline: 5.7874x; 5.7874x over previous
import functools

import jax
import jax.numpy as jnp
from jax import lax
from jax.experimental import pallas as pl
from jax.experimental.pallas import tpu as pltpu
from jax.experimental.pallas import tpu_sc as plsc

N = 16384 * 512
NW = 32
CW = N // NW
C = 16384
T = CW // C
NV = C // 16
NCV = (NW * T) // 16


def _worker_id():
    return lax.axis_index("s") * 2 + lax.axis_index("c")


def _count_kernel(mask_hbm, counts_hbm, mbuf, cvec):
    wid = _worker_id()
    base = wid * CW
    lane = lax.iota(jnp.int32, 16)

    def sub(t, counts_vec):
        pltpu.sync_copy(mask_hbm.at[pl.ds(base + t * C, C)], mbuf)

        def inner(i, acc):
            return acc + mbuf[pl.ds(i * 16, 16)]

        acc = lax.fori_loop(0, NV, inner, jnp.zeros((16,), jnp.int32))
        cnt = jnp.sum(acc)
        return jnp.where(lane == t, cnt, counts_vec)

    counts_vec = lax.fori_loop(0, T, sub, jnp.zeros((16,), jnp.int32))
    cvec[...] = counts_vec
    pltpu.sync_copy(cvec, counts_hbm.at[pl.ds(wid * 16, 16)])


def _scatter_kernel(mask_hbm, input_hbm, source_hbm, counts_hbm, out_hbm,
                    mbuf, ibuf, sbuf, obuf, cbuf):
    wid = _worker_id()
    lane = lax.iota(jnp.int32, 16)
    pltpu.sync_copy(counts_hbm, cbuf)

    lim = wid * T

    def accw(j, s):
        vec = cbuf[pl.ds(j * 16, 16)]
        gidx = j * 16 + lane
        return s + jnp.sum(jnp.where(gidx < lim, vec, 0))

    base0 = lax.fori_loop(0, NCV, accw, jnp.int32(0))

    def sub(t, base):
        eb = wid * CW + t * C
        pltpu.sync_copy(mask_hbm.at[pl.ds(eb, C)], mbuf)
        pltpu.sync_copy(input_hbm.at[pl.ds(eb, C)], ibuf)
        j = wid * T + t
        cvec = cbuf[pl.ds((j // 16) * 16, 16)]
        cnt = jnp.sum(jnp.where(lane == (j % 16), cvec, 0))
        start = jnp.minimum((base // 8) * 8, N - (C + 8))
        pltpu.sync_copy(source_hbm.at[pl.ds(start, C + 8)], sbuf)
        off = base - start

        def inner(i, carry):
            m = mbuf[pl.ds(i * 16, 16)]
            inc = plsc.cumsum(m)
            idxv = jnp.full((16,), carry, jnp.int32) + (inc - m)
            vals = plsc.load_gather(sbuf, [idxv])
            iv = ibuf[pl.ds(i * 16, 16)]
            obuf[pl.ds(i * 16, 16)] = jnp.where(m != 0, vals, iv)
            return carry + jnp.sum(m)

        lax.fori_loop(0, NV, inner, off)
        pltpu.sync_copy(obuf, out_hbm.at[pl.ds(eb, C)])
        return base + cnt

    lax.fori_loop(0, T, sub, base0)


def kernel(input, mask, source):
    shape = input.shape
    flat_in = input.reshape(-1)
    flat_src = source.reshape(-1)
    flat_mask = mask.reshape(-1).astype(jnp.int32)

    mesh = plsc.VectorSubcoreMesh(core_axis_name="c", subcore_axis_name="s")

    params = pltpu.CompilerParams(needs_layout_passes=False)

    count_call = functools.partial(
        pl.kernel,
        mesh=mesh,
        compiler_params=params,
        out_type=jax.ShapeDtypeStruct((NW * T,), jnp.int32),
        scratch_types=[
            pltpu.VMEM((C,), jnp.int32),
            pltpu.VMEM((16,), jnp.int32),
        ],
    )(_count_kernel)
    counts = count_call(flat_mask)

    scatter_call = functools.partial(
        pl.kernel,
        mesh=mesh,
        compiler_params=params,
        out_type=jax.ShapeDtypeStruct((N,), jnp.float32),
        scratch_types=[
            pltpu.VMEM((C,), jnp.int32),
            pltpu.VMEM((C,), jnp.float32),
            pltpu.VMEM((C + 8,), jnp.float32),
            pltpu.VMEM((C,), jnp.float32),
            pltpu.VMEM((NW * T,), jnp.int32),
        ],
    )(_scatter_kernel)
    out = scatter_call(flat_mask, flat_in, flat_src, counts)
    return out.reshape(shape)
